# Initial kernel scaffold; baseline (speedup 1.0000x reference)
import jax
import jax.numpy as jnp
from jax.experimental import pallas as pl


def kernel(p_0, c_0, v_0, e, t, W_in, b_in, We1, be1, We2, be2, Wa, ba, Wx1, bx1, Wx2, Wh1, bh1, Wh2, bh2, W_out, b_out, position_scale):
    raise NotImplementedError("write your pallas kernel here")


# speedup vs baseline: 5.2607x; 5.2607x over previous
import functools

import jax
import jax.numpy as jnp
from jax import lax
from jax.experimental import pallas as pl
from jax.experimental.pallas import tpu as pltpu
from jax.experimental.pallas import tpu_sc as plsc

NUM_STEPS = 100
K = 20
HID = 20
NL = 4

LN = 50000
LP = 50176
EN = 800000
EP = 802816
BL = 3584
BE = 8192
NCHG = 392
NCHS = 196
NB = 4
TPS = 3136
ZCH = 392
ZB_R = 196

F32 = jnp.float32


def _silu(x):
    return x * jax.nn.sigmoid(x)


def _sc_gather(tr, tc, gidx):
    mesh = plsc.VectorSubcoreMesh(core_axis_name="c", subcore_axis_name="s")

    def body(tr_ref, tc_ref, gi_ref, out_ref, idx_v, bufs, gsem, wsem):
        c_id = lax.axis_index("c")
        s_id = lax.axis_index("s")
        wid = c_id * 16 + s_id
        pltpu.sync_copy(gi_ref.at[wid], idx_v)
        base = wid * (NCHG * 128)

        def run(tbl):
            for b in range(NB):
                pltpu.async_copy(tbl.at[idx_v.at[b]], bufs.at[b], gsem.at[b])

            def step(g, carry):
                for b in range(NB):
                    c = g * NB + b
                    pltpu.make_async_copy(
                        tbl.at[idx_v.at[c]], bufs.at[b], gsem.at[b]).wait()
                    pltpu.async_copy(
                        bufs.at[b], out_ref.at[pl.ds(base + c * 128, 128)],
                        wsem.at[b])
                    nxt = c + NB

                    @pl.when(nxt < NCHG)
                    def _():
                        pltpu.make_async_copy(
                            bufs.at[b], out_ref.at[pl.ds(base, 128)],
                            wsem.at[b]).wait()
                        pltpu.async_copy(
                            tbl.at[idx_v.at[nxt]], bufs.at[b], gsem.at[b])
                return carry

            lax.fori_loop(0, NCHG // NB, step, 0)
            for b in range(NB):
                pltpu.make_async_copy(
                    bufs.at[b], out_ref.at[pl.ds(base, 128)], wsem.at[b]).wait()

        @pl.when(c_id == 0)
        def _():
            run(tr_ref)

        @pl.when(c_id == 1)
        def _():
            run(tc_ref)

    f = pl.kernel(
        body,
        out_type=jax.ShapeDtypeStruct((2 * EP, 32), F32),
        mesh=mesh,
        compiler_params=pltpu.CompilerParams(use_tc_tiling_on_sc=False),
        scratch_types=[
            pltpu.VMEM((NCHG, 128), jnp.int32),
            pltpu.VMEM((NB, 128, 32), F32),
            pltpu.SemaphoreType.DMA((NB,)),
            pltpu.SemaphoreType.DMA((NB,)),
        ],
    )
    return f(tr, tc, gidx)


def _sc_scatter(s_vals, sidx):
    mesh = plsc.VectorSubcoreMesh(core_axis_name="c", subcore_axis_name="s")

    def body(s_ref, si_ref, out_ref, acc, iring, vals, zb, lsem, isem):
        c_id = lax.axis_index("c")
        s_id = lax.axis_index("s")
        wid = c_id * 16 + s_id

        def zfill(r, carry):
            zb[r, pl.ds(0, 16)] = jnp.zeros((16,), F32)
            zb[r, pl.ds(16, 16)] = jnp.zeros((16,), F32)
            return carry

        lax.fori_loop(0, ZB_R, zfill, 0)
        for i in range(TPS // ZB_R):
            pltpu.sync_copy(zb, acc.at[pl.ds(s_id * TPS + i * ZB_R, ZB_R)])
        plsc.subcore_barrier()

        base = wid * (NCHS * 128)
        for b in range(NB):
            pltpu.async_copy(
                s_ref.at[pl.ds(base + b * 128, 128)], vals.at[b], lsem.at[b])
            pltpu.async_copy(si_ref.at[wid, b], iring.at[b], isem.at[b])

        def step(g, carry):
            for b in range(NB):
                c = g * NB + b
                pltpu.make_async_copy(
                    s_ref.at[pl.ds(base, 128)], vals.at[b], lsem.at[b]).wait()
                pltpu.make_async_copy(
                    si_ref.at[wid, c], iring.at[b], isem.at[b]).wait()
                pltpu.sync_copy(vals.at[b], acc.at[iring.at[b]], add=True)
                nxt = c + NB

                @pl.when(nxt < NCHS)
                def _():
                    pltpu.async_copy(
                        s_ref.at[pl.ds(base + nxt * 128, 128)], vals.at[b],
                        lsem.at[b])
                    pltpu.async_copy(si_ref.at[wid, nxt], iring.at[b],
                                     isem.at[b])
            return carry

        lax.fori_loop(0, NCHS // NB, step, 0)
        plsc.subcore_barrier()
        for i in range(TPS // ZCH):
            off = s_id * TPS + i * ZCH
            pltpu.sync_copy(acc.at[pl.ds(off, ZCH)],
                            out_ref.at[c_id, pl.ds(off, ZCH)])

    f = pl.kernel(
        body,
        out_type=jax.ShapeDtypeStruct((2, LP, 32), F32),
        mesh=mesh,
        compiler_params=pltpu.CompilerParams(use_tc_tiling_on_sc=False),
        scratch_types=[
            pltpu.VMEM_SHARED((LP, 32), F32),
            pltpu.VMEM((NB, 128), jnp.int32),
            pltpu.VMEM((NB, 128, 32), F32),
            pltpu.VMEM((ZB_R, 32), F32),
            pltpu.SemaphoreType.DMA((NB,)),
            pltpu.SemaphoreType.DMA((NB,)),
        ],
    )
    return f(s_vals, sidx)


def _tc_init(cn, x0, win_a, b_eff, a0, b0, interpret=False):
    def body(cn_ref, x_ref, wa_ref, be_ref, a_ref, b_ref,
             h_ref, tr_ref, tc_ref):
        h = jnp.dot(cn_ref[...], wa_ref[...],
                    preferred_element_type=F32) + be_ref[...]
        x = x_ref[...]
        p = jnp.dot(h, a_ref[...], preferred_element_type=F32)
        q = jnp.dot(h, b_ref[...], preferred_element_type=F32)
        z = jnp.zeros((BL, 3), F32)
        h_ref[...] = h
        tr_ref[...] = jnp.concatenate([p, x, z], axis=-1)
        tc_ref[...] = jnp.concatenate([q, x, z], axis=-1)

    grid = LP // BL
    return pl.pallas_call(
        body,
        grid=(grid,),
        in_specs=[
            pl.BlockSpec((BL, HID), lambda i: (i, 0)),
            pl.BlockSpec((BL, 9), lambda i: (i, 0)),
            pl.BlockSpec((HID, HID), lambda i: (0, 0)),
            pl.BlockSpec((1, HID), lambda i: (0, 0)),
            pl.BlockSpec((HID, HID), lambda i: (0, 0)),
            pl.BlockSpec((HID, HID), lambda i: (0, 0)),
        ],
        out_specs=[
            pl.BlockSpec((BL, HID), lambda i: (i, 0)),
            pl.BlockSpec((BL, 32), lambda i: (i, 0)),
            pl.BlockSpec((BL, 32), lambda i: (i, 0)),
        ],
        out_shape=[
            jax.ShapeDtypeStruct((LP, HID), F32),
            jax.ShapeDtypeStruct((LP, 32), F32),
            jax.ShapeDtypeStruct((LP, 32), F32),
        ],
        interpret=interpret,
    )(cn, x0, win_a, b_eff, a0, b0)


def _tc_mlp(g, we1c, be1, we2, be2, wa_t, ba, wx1, bx1, wx2_t, interpret=False):
    def body(gr_ref, gc_ref, c_ref, b1_ref, w2_ref, b2_ref, wa_ref, ba_ref,
             wx1_ref, bx1_ref, wx2_ref, s_ref):
        gr = gr_ref[...]
        gc = gc_ref[...]
        diff = gr[:, 20:29] - gc[:, 20:29]
        radial = jnp.sum(diff * diff, axis=-1, keepdims=True)
        m1 = gr[:, 0:20] + gc[:, 0:20] + radial * c_ref[...] + b1_ref[...]
        m1 = _silu(m1)
        m2 = _silu(jnp.dot(m1, w2_ref[...],
                           preferred_element_type=F32) + b2_ref[...])
        att = jax.nn.sigmoid(
            jnp.sum(m2 * wa_ref[...], axis=-1, keepdims=True) + ba_ref[...])
        m = m2 * att
        u = _silu(jnp.dot(m, wx1_ref[...],
                          preferred_element_type=F32) + bx1_ref[...])
        w = jnp.sum(u * wx2_ref[...], axis=-1, keepdims=True)
        trans = diff * w
        s_ref[...] = jnp.concatenate(
            [m, trans, jnp.ones((BE, 1), F32), jnp.zeros((BE, 2), F32)],
            axis=-1)

    grid = EP // BE
    wfull = lambda shape: pl.BlockSpec(shape, lambda i: (0, 0))
    return pl.pallas_call(
        body,
        grid=(grid,),
        in_specs=[
            pl.BlockSpec((BE, 32), lambda i: (i, 0)),
            pl.BlockSpec((BE, 32), lambda i: (i + EP // BE, 0)),
            wfull((1, HID)),
            wfull((1, HID)),
            wfull((HID, HID)),
            wfull((1, HID)),
            wfull((1, HID)),
            wfull((1, 1)),
            wfull((HID, HID)),
            wfull((1, HID)),
            wfull((1, HID)),
        ],
        out_specs=pl.BlockSpec((BE, 32), lambda i: (i, 0)),
        out_shape=jax.ShapeDtypeStruct((EP, 32), F32),
        interpret=interpret,
    )(g, g, we1c, be1, we2, be2, wa_t, ba, wx1, bx1, wx2_t)


def _tc_node(h, x, acc, wh1a, wh1b, bh1, wh2, bh2, a_nxt, b_nxt,
             interpret=False):
    def body(h_ref, x_ref, a0_ref, a1_ref, wh1a_ref, wh1b_ref, bh1_ref,
             wh2_ref, bh2_ref, an_ref, bn_ref,
             ho_ref, xo_ref, tro_ref, tco_ref):
        accs = a0_ref[0] + a1_ref[0]
        agg_m = accs[:, 0:20]
        cnt = jnp.maximum(accs[:, 29:30], 1.0)
        agg_x = accs[:, 20:29] / cnt
        x_new = x_ref[...] + agg_x
        h_old = h_ref[...]
        pre = jnp.dot(h_old, wh1a_ref[...], preferred_element_type=F32)
        pre = pre + jnp.dot(agg_m, wh1b_ref[...], preferred_element_type=F32)
        hn = jnp.dot(_silu(pre + bh1_ref[...]), wh2_ref[...],
                     preferred_element_type=F32) + bh2_ref[...]
        h_new = h_old + hn
        p = jnp.dot(h_new, an_ref[...], preferred_element_type=F32)
        q = jnp.dot(h_new, bn_ref[...], preferred_element_type=F32)
        z = jnp.zeros((BL, 3), F32)
        ho_ref[...] = h_new
        xo_ref[...] = x_new
        tro_ref[...] = jnp.concatenate([p, x_new, z], axis=-1)
        tco_ref[...] = jnp.concatenate([q, x_new, z], axis=-1)

    grid = LP // BL
    wfull = lambda shape: pl.BlockSpec(shape, lambda i: (0, 0))
    return pl.pallas_call(
        body,
        grid=(grid,),
        in_specs=[
            pl.BlockSpec((BL, HID), lambda i: (i, 0)),
            pl.BlockSpec((BL, 9), lambda i: (i, 0)),
            pl.BlockSpec((1, BL, 32), lambda i: (0, i, 0)),
            pl.BlockSpec((1, BL, 32), lambda i: (1, i, 0)),
            wfull((HID, HID)),
            wfull((HID, HID)),
            wfull((1, HID)),
            wfull((HID, HID)),
            wfull((1, HID)),
            wfull((HID, HID)),
            wfull((HID, HID)),
        ],
        out_specs=[
            pl.BlockSpec((BL, HID), lambda i: (i, 0)),
            pl.BlockSpec((BL, 9), lambda i: (i, 0)),
            pl.BlockSpec((BL, 32), lambda i: (i, 0)),
            pl.BlockSpec((BL, 32), lambda i: (i, 0)),
        ],
        out_shape=[
            jax.ShapeDtypeStruct((LP, HID), F32),
            jax.ShapeDtypeStruct((LP, 9), F32),
            jax.ShapeDtypeStruct((LP, 32), F32),
            jax.ShapeDtypeStruct((LP, 32), F32),
        ],
        interpret=interpret,
    )(h, x, acc, acc, wh1a, wh1b, bh1, wh2, bh2, a_nxt, b_nxt)


def _tc_loss(h, x, pn, ep, ct, c0h, w_out, b_out, sc, interpret=False):
    def body(h_ref, x_ref, pn_ref, ep_ref, ct_ref, c0h_ref, wo_ref, bo_ref,
             sc_ref, lp_ref, ls_ref):
        i = pl.program_id(0)
        cd = jax.nn.softmax(
            jnp.dot(h_ref[...], wo_ref[...],
                    preferred_element_type=F32) + bo_ref[...], axis=-1)
        alpha = sc_ref[0, 0]
        abp = sc_ref[0, 1]
        fac = alpha * ct_ref[...] + (1.0 - alpha) / K
        tt = fac * (abp * c0h_ref[...] + (1.0 - abp) / K)
        tp = fac * (abp * cd + (1.0 - abp) / K)
        pt = tt / jnp.sum(tt, axis=-1, keepdims=True)
        pp = tp / jnp.sum(tp, axis=-1, keepdims=True)
        kl = jnp.sum(pt * jnp.log(pt) - pt * jnp.log(pp + 1e-8), axis=-1,
                     keepdims=True)
        rows = i * BL + lax.broadcasted_iota(jnp.int32, (BL, 1), 0)
        msk = (rows < LN).astype(F32)
        d = (x_ref[...] - pn_ref[...]) - ep_ref[...]
        lpos = jnp.sum(jnp.sum(d * d, axis=-1, keepdims=True) * msk)
        lseq = jnp.sum(kl * msk)

        @pl.when(i == 0)
        def _():
            lp_ref[...] = jnp.zeros((1, 1), F32)
            ls_ref[...] = jnp.zeros((1, 1), F32)

        lp_ref[...] = lp_ref[...] + jnp.reshape(lpos, (1, 1))
        ls_ref[...] = ls_ref[...] + jnp.reshape(lseq, (1, 1))

    grid = LP // BL
    wfull = lambda shape: pl.BlockSpec(shape, lambda i: (0, 0))
    return pl.pallas_call(
        body,
        grid=(grid,),
        in_specs=[
            pl.BlockSpec((BL, HID), lambda i: (i, 0)),
            pl.BlockSpec((BL, 9), lambda i: (i, 0)),
            pl.BlockSpec((BL, 9), lambda i: (i, 0)),
            pl.BlockSpec((BL, 9), lambda i: (i, 0)),
            pl.BlockSpec((BL, K), lambda i: (i, 0)),
            pl.BlockSpec((BL, K), lambda i: (i, 0)),
            wfull((HID, K)),
            wfull((1, K)),
            wfull((1, 2)),
        ],
        out_specs=[
            pl.BlockSpec((1, 1), lambda i: (0, 0)),
            pl.BlockSpec((1, 1), lambda i: (0, 0)),
        ],
        out_shape=[
            jax.ShapeDtypeStruct((1, 1), F32),
            jax.ShapeDtypeStruct((1, 1), F32),
        ],
        interpret=interpret,
    )(h, x, pn, ep, ct, c0h, w_out, b_out, sc)


def _pad_rows(a, n):
    return jnp.concatenate(
        [a, jnp.zeros((n - a.shape[0],) + a.shape[1:], a.dtype)], axis=0)


def kernel(p_0, c_0, v_0, e, t, W_in, b_in, We1, be1, We2, be2, Wa, ba,
           Wx1, bx1, Wx2, Wh1, bh1, Wh2, bh2, W_out, b_out, position_scale):
    betas = jnp.linspace(1e-4, 0.02, NUM_STEPS)
    alphas = 1.0 - betas
    alpha_bars = jnp.cumprod(alphas)

    row = e[0, 0].astype(jnp.int32)
    col = e[1, 0].astype(jnp.int32)

    p_norm = p_0 / position_scale
    key = jax.random.key(42)
    k1, k2, k3 = jax.random.split(key, 3)
    ab = alpha_bars[t]
    c0f = jnp.sqrt(ab)[:, None, None]
    c1f = jnp.sqrt(1.0 - ab)[:, None, None]
    eps_p = jax.random.normal(k1, p_norm.shape, dtype=p_norm.dtype)
    p_noisy = c0f * p_norm + c1f * eps_p
    s_0 = jax.random.categorical(k2, jnp.log(c_0 + 1e-8), axis=-1)
    c0h = jax.nn.one_hot(s_0, K, dtype=F32)
    c_noisy = ab[:, None, None] * c0h + (1.0 - ab[:, None, None]) / K
    s_noisy = jax.random.categorical(k3, jnp.log(c_noisy + 1e-8), axis=-1)
    beta = betas[t]

    te = jnp.stack([beta, jnp.sin(beta), jnp.cos(beta)], axis=-1)
    b_eff = te @ W_in[20:23] + b_in[None, :]
    win_a = W_in[0:20]

    alpha_t = alphas[t]
    ab_prev = jnp.where(t > 0, alpha_bars[jnp.maximum(t - 1, 0)], 1.0)
    sc = jnp.stack([alpha_t[0], ab_prev[0]]).reshape(1, 2).astype(F32)

    cn = _pad_rows(c_noisy[0], LP)
    x0 = _pad_rows(p_noisy[0], LP)
    pn = x0
    ep = _pad_rows(eps_p[0], LP)
    ct = _pad_rows(jax.nn.one_hot(s_noisy[0], K, dtype=F32), LP)
    c0hp = _pad_rows(c0h[0], LP)

    zpad = jnp.zeros((EP - EN,), jnp.int32)
    gidx = jnp.concatenate([row, zpad, col, zpad]).reshape(32, NCHG, 128)
    sidx = jnp.concatenate(
        [row, jnp.full((EP - EN,), LN, jnp.int32)]).reshape(32, NCHS, 128)

    a_l = [We1[l, 0:20, :] for l in range(NL)]
    b_l = [We1[l, 20:40, :] for l in range(NL)]
    we1c_l = [We1[l, 40:41, :] for l in range(NL)]
    be1_l = [be1[l][None, :] for l in range(NL)]
    be2_l = [be2[l][None, :] for l in range(NL)]
    wa_l = [Wa[l].T for l in range(NL)]
    ba_l = [ba[l].reshape(1, 1) for l in range(NL)]
    bx1_l = [bx1[l][None, :] for l in range(NL)]
    wx2_l = [Wx2[l].T for l in range(NL)]
    wh1a_l = [Wh1[l, 0:20, :] for l in range(NL)]
    wh1b_l = [Wh1[l, 20:40, :] for l in range(NL)]
    bh1_l = [bh1[l][None, :] for l in range(NL)]
    bh2_l = [bh2[l][None, :] for l in range(NL)]

    h, tr, tcn = _tc_init(cn, x0, win_a, b_eff, a_l[0], b_l[0])
    x = x0
    for l in range(NL):
        g = _sc_gather(tr, tcn, gidx)
        s_vals = _tc_mlp(g, we1c_l[l], be1_l[l], We2[l], be2_l[l], wa_l[l],
                         ba_l[l], Wx1[l], bx1_l[l], wx2_l[l])
        acc = _sc_scatter(s_vals, sidx)
        nxt = (l + 1) % NL
        h, x, tr, tcn = _tc_node(h, x, acc, wh1a_l[l], wh1b_l[l], bh1_l[l],
                                 Wh2[l], bh2_l[l], a_l[nxt], b_l[nxt])

    lp, ls = _tc_loss(h, x, pn, ep, ct, c0hp, W_out, b_out[None, :], sc)
    denom = jnp.float32(LN) + 1e-8
    return (lp[0, 0] / denom, ls[0, 0] / denom)


# speedup vs baseline: 6.8981x; 1.3112x over previous
import functools

import jax
import jax.numpy as jnp
from jax import lax
from jax.experimental import pallas as pl
from jax.experimental.pallas import tpu as pltpu
from jax.experimental.pallas import tpu_sc as plsc

NUM_STEPS = 100
K = 20
HID = 20
NL = 4

LN = 50000
LP = 50176
EN = 800000
EP = 802816
BL = 3584
BE = 8192
NCHG = 392
NCHS = 196
NB = 4
TPS = 3136
ZCH = 392
ZB_R = 196

F32 = jnp.float32


def _silu(x):
    return x * jax.nn.sigmoid(x)


def _sc_gather(tr, tc, gidx):
    mesh = plsc.VectorSubcoreMesh(core_axis_name="c", subcore_axis_name="s")

    def body(tr_ref, tc_ref, gi_ref, out_ref, idx_v, bufs, gsem, wsem):
        c_id = lax.axis_index("c")
        s_id = lax.axis_index("s")
        wid = c_id * 16 + s_id
        pltpu.sync_copy(gi_ref.at[wid], idx_v)
        base = wid * (NCHG * 128)

        def run(tbl):
            for b in range(NB):
                pltpu.async_copy(tbl.at[idx_v.at[b]], bufs.at[b], gsem.at[b])

            def step(g, carry):
                for b in range(NB):
                    c = g * NB + b
                    pltpu.make_async_copy(
                        tbl.at[idx_v.at[c]], bufs.at[b], gsem.at[b]).wait()
                    pltpu.async_copy(
                        bufs.at[b], out_ref.at[pl.ds(base + c * 128, 128)],
                        wsem.at[b])
                    nxt = c + NB

                    @pl.when(nxt < NCHG)
                    def _():
                        pltpu.make_async_copy(
                            bufs.at[b], out_ref.at[pl.ds(base, 128)],
                            wsem.at[b]).wait()
                        pltpu.async_copy(
                            tbl.at[idx_v.at[nxt]], bufs.at[b], gsem.at[b])
                return carry

            lax.fori_loop(0, NCHG // NB, step, 0)
            for b in range(NB):
                pltpu.make_async_copy(
                    bufs.at[b], out_ref.at[pl.ds(base, 128)], wsem.at[b]).wait()

        @pl.when(c_id == 0)
        def _():
            run(tr_ref)

        @pl.when(c_id == 1)
        def _():
            run(tc_ref)

    f = pl.kernel(
        body,
        out_type=jax.ShapeDtypeStruct((2 * EP, 32), F32),
        mesh=mesh,
        compiler_params=pltpu.CompilerParams(use_tc_tiling_on_sc=False),
        scratch_types=[
            pltpu.VMEM((NCHG, 128), jnp.int32),
            pltpu.VMEM((NB, 128, 32), F32),
            pltpu.SemaphoreType.DMA((NB,)),
            pltpu.SemaphoreType.DMA((NB,)),
        ],
    )
    return f(tr, tc, gidx)


def _sc_scatter(s_vals, sidx):
    mesh = plsc.VectorSubcoreMesh(core_axis_name="c", subcore_axis_name="s")

    def body(s_ref, si_ref, out_ref, acc, iring, vals, zb, lsem, isem):
        c_id = lax.axis_index("c")
        s_id = lax.axis_index("s")
        wid = c_id * 16 + s_id

        def zfill(r, carry):
            zb[r, pl.ds(0, 16)] = jnp.zeros((16,), F32)
            zb[r, pl.ds(16, 16)] = jnp.zeros((16,), F32)
            return carry

        lax.fori_loop(0, ZB_R, zfill, 0)
        for i in range(TPS // ZB_R):
            pltpu.sync_copy(zb, acc.at[pl.ds(s_id * TPS + i * ZB_R, ZB_R)])
        plsc.subcore_barrier()

        base = wid * (NCHS * 128)
        for b in range(NB):
            pltpu.async_copy(
                s_ref.at[pl.ds(base + b * 128, 128)], vals.at[b], lsem.at[b])
            pltpu.async_copy(si_ref.at[wid, b], iring.at[b], isem.at[b])

        def step(g, carry):
            for b in range(NB):
                c = g * NB + b
                pltpu.make_async_copy(
                    s_ref.at[pl.ds(base, 128)], vals.at[b], lsem.at[b]).wait()
                pltpu.make_async_copy(
                    si_ref.at[wid, c], iring.at[b], isem.at[b]).wait()
                pltpu.sync_copy(vals.at[b], acc.at[iring.at[b]], add=True)
                nxt = c + NB

                @pl.when(nxt < NCHS)
                def _():
                    pltpu.async_copy(
                        s_ref.at[pl.ds(base + nxt * 128, 128)], vals.at[b],
                        lsem.at[b])
                    pltpu.async_copy(si_ref.at[wid, nxt], iring.at[b],
                                     isem.at[b])
            return carry

        lax.fori_loop(0, NCHS // NB, step, 0)
        plsc.subcore_barrier()
        for i in range(TPS // ZCH):
            off = s_id * TPS + i * ZCH
            pltpu.sync_copy(acc.at[pl.ds(off, ZCH)],
                            out_ref.at[c_id, pl.ds(off, ZCH)])

    f = pl.kernel(
        body,
        out_type=jax.ShapeDtypeStruct((2, LP, 32), F32),
        mesh=mesh,
        compiler_params=pltpu.CompilerParams(use_tc_tiling_on_sc=False),
        scratch_types=[
            pltpu.VMEM_SHARED((LP, 32), F32),
            pltpu.VMEM((NB, 128), jnp.int32),
            pltpu.VMEM((NB, 128, 32), F32),
            pltpu.VMEM((ZB_R, 32), F32),
            pltpu.SemaphoreType.DMA((NB,)),
            pltpu.SemaphoreType.DMA((NB,)),
        ],
    )
    return f(s_vals, sidx)


def _tc_init(cn, x0, win_a, b_eff, a0, b0, interpret=False):
    def body(cn_ref, x_ref, wa_ref, be_ref, a_ref, b_ref,
             h_ref, tr_ref, tc_ref):
        h = jnp.dot(cn_ref[...], wa_ref[...],
                    preferred_element_type=F32) + be_ref[...]
        x = x_ref[...]
        p = jnp.dot(h, a_ref[...], preferred_element_type=F32)
        q = jnp.dot(h, b_ref[...], preferred_element_type=F32)
        z = jnp.zeros((BL, 3), F32)
        h_ref[...] = h
        tr_ref[...] = jnp.concatenate([p, x, z], axis=-1)
        tc_ref[...] = jnp.concatenate([q, x, z], axis=-1)

    grid = LP // BL
    return pl.pallas_call(
        body,
        grid=(grid,),
        in_specs=[
            pl.BlockSpec((BL, HID), lambda i: (i, 0)),
            pl.BlockSpec((BL, 9), lambda i: (i, 0)),
            pl.BlockSpec((HID, HID), lambda i: (0, 0)),
            pl.BlockSpec((1, HID), lambda i: (0, 0)),
            pl.BlockSpec((HID, HID), lambda i: (0, 0)),
            pl.BlockSpec((HID, HID), lambda i: (0, 0)),
        ],
        out_specs=[
            pl.BlockSpec((BL, HID), lambda i: (i, 0)),
            pl.BlockSpec((BL, 32), lambda i: (i, 0)),
            pl.BlockSpec((BL, 32), lambda i: (i, 0)),
        ],
        out_shape=[
            jax.ShapeDtypeStruct((LP, HID), F32),
            jax.ShapeDtypeStruct((LP, 32), F32),
            jax.ShapeDtypeStruct((LP, 32), F32),
        ],
        interpret=interpret,
    )(cn, x0, win_a, b_eff, a0, b0)


def _tc_mlp(g4, we1c, be1, we2, be2, wa_t, ba, wx1, bx1, wx2_t,
            interpret=False):
    B4 = BE // 4

    def body(gr_ref, gc_ref, c_ref, b1_ref, w2_ref, b2_ref, wa_ref, ba_ref,
             wx1_ref, bx1_ref, wx2_ref, s_ref):
        gr4 = gr_ref[...]
        gc4 = gc_ref[...]
        for k in range(4):
            gr = gr4[:, 32 * k:32 * k + 32]
            gc = gc4[:, 32 * k:32 * k + 32]
            diff = gr[:, 20:29] - gc[:, 20:29]
            radial = jnp.sum(diff * diff, axis=-1, keepdims=True)
            m1 = gr[:, 0:20] + gc[:, 0:20] + radial * c_ref[...] + b1_ref[...]
            m1 = _silu(m1)
            m2 = _silu(jnp.dot(m1, w2_ref[...],
                               preferred_element_type=F32) + b2_ref[...])
            att = jax.nn.sigmoid(
                jnp.sum(m2 * wa_ref[...], axis=-1, keepdims=True) + ba_ref[...])
            m = m2 * att
            u = _silu(jnp.dot(m, wx1_ref[...],
                              preferred_element_type=F32) + bx1_ref[...])
            w = jnp.sum(u * wx2_ref[...], axis=-1, keepdims=True)
            trans = diff * w
            s_ref[:, 32 * k:32 * k + 32] = jnp.concatenate(
                [m, trans, jnp.ones((B4, 1), F32), jnp.zeros((B4, 2), F32)],
                axis=-1)

    grid = EP // BE
    wfull = lambda shape: pl.BlockSpec(shape, lambda i: (0, 0))
    return pl.pallas_call(
        body,
        grid=(grid,),
        in_specs=[
            pl.BlockSpec((B4, 128), lambda i: (i, 0)),
            pl.BlockSpec((B4, 128), lambda i: (i + EP // BE, 0)),
            wfull((1, HID)),
            wfull((1, HID)),
            wfull((HID, HID)),
            wfull((1, HID)),
            wfull((1, HID)),
            wfull((1, 1)),
            wfull((HID, HID)),
            wfull((1, HID)),
            wfull((1, HID)),
        ],
        out_specs=pl.BlockSpec((B4, 128), lambda i: (i, 0)),
        out_shape=jax.ShapeDtypeStruct((EP // 4, 128), F32),
        interpret=interpret,
    )(g4, g4, we1c, be1, we2, be2, wa_t, ba, wx1, bx1, wx2_t)


def _tc_node(h, x, acc, wh1a, wh1b, bh1, wh2, bh2, a_nxt, b_nxt,
             interpret=False):
    def body(h_ref, x_ref, a0_ref, a1_ref, wh1a_ref, wh1b_ref, bh1_ref,
             wh2_ref, bh2_ref, an_ref, bn_ref,
             ho_ref, xo_ref, tro_ref, tco_ref):
        accs = a0_ref[0] + a1_ref[0]
        agg_m = accs[:, 0:20]
        cnt = jnp.maximum(accs[:, 29:30], 1.0)
        agg_x = accs[:, 20:29] / cnt
        x_new = x_ref[...] + agg_x
        h_old = h_ref[...]
        pre = jnp.dot(h_old, wh1a_ref[...], preferred_element_type=F32)
        pre = pre + jnp.dot(agg_m, wh1b_ref[...], preferred_element_type=F32)
        hn = jnp.dot(_silu(pre + bh1_ref[...]), wh2_ref[...],
                     preferred_element_type=F32) + bh2_ref[...]
        h_new = h_old + hn
        p = jnp.dot(h_new, an_ref[...], preferred_element_type=F32)
        q = jnp.dot(h_new, bn_ref[...], preferred_element_type=F32)
        z = jnp.zeros((BL, 3), F32)
        ho_ref[...] = h_new
        xo_ref[...] = x_new
        tro_ref[...] = jnp.concatenate([p, x_new, z], axis=-1)
        tco_ref[...] = jnp.concatenate([q, x_new, z], axis=-1)

    grid = LP // BL
    wfull = lambda shape: pl.BlockSpec(shape, lambda i: (0, 0))
    return pl.pallas_call(
        body,
        grid=(grid,),
        in_specs=[
            pl.BlockSpec((BL, HID), lambda i: (i, 0)),
            pl.BlockSpec((BL, 9), lambda i: (i, 0)),
            pl.BlockSpec((1, BL, 32), lambda i: (0, i, 0)),
            pl.BlockSpec((1, BL, 32), lambda i: (1, i, 0)),
            wfull((HID, HID)),
            wfull((HID, HID)),
            wfull((1, HID)),
            wfull((HID, HID)),
            wfull((1, HID)),
            wfull((HID, HID)),
            wfull((HID, HID)),
        ],
        out_specs=[
            pl.BlockSpec((BL, HID), lambda i: (i, 0)),
            pl.BlockSpec((BL, 9), lambda i: (i, 0)),
            pl.BlockSpec((BL, 32), lambda i: (i, 0)),
            pl.BlockSpec((BL, 32), lambda i: (i, 0)),
        ],
        out_shape=[
            jax.ShapeDtypeStruct((LP, HID), F32),
            jax.ShapeDtypeStruct((LP, 9), F32),
            jax.ShapeDtypeStruct((LP, 32), F32),
            jax.ShapeDtypeStruct((LP, 32), F32),
        ],
        interpret=interpret,
    )(h, x, acc, acc, wh1a, wh1b, bh1, wh2, bh2, a_nxt, b_nxt)


def _tc_loss(h, x, pn, ep, ct, c0h, w_out, b_out, sc, interpret=False):
    def body(h_ref, x_ref, pn_ref, ep_ref, ct_ref, c0h_ref, wo_ref, bo_ref,
             sc_ref, lp_ref, ls_ref):
        i = pl.program_id(0)
        cd = jax.nn.softmax(
            jnp.dot(h_ref[...], wo_ref[...],
                    preferred_element_type=F32) + bo_ref[...], axis=-1)
        alpha = sc_ref[0, 0]
        abp = sc_ref[0, 1]
        fac = alpha * ct_ref[...] + (1.0 - alpha) / K
        tt = fac * (abp * c0h_ref[...] + (1.0 - abp) / K)
        tp = fac * (abp * cd + (1.0 - abp) / K)
        pt = tt / jnp.sum(tt, axis=-1, keepdims=True)
        pp = tp / jnp.sum(tp, axis=-1, keepdims=True)
        kl = jnp.sum(pt * jnp.log(pt) - pt * jnp.log(pp + 1e-8), axis=-1,
                     keepdims=True)
        rows = i * BL + lax.broadcasted_iota(jnp.int32, (BL, 1), 0)
        msk = (rows < LN).astype(F32)
        d = (x_ref[...] - pn_ref[...]) - ep_ref[...]
        lpos = jnp.sum(jnp.sum(d * d, axis=-1, keepdims=True) * msk)
        lseq = jnp.sum(kl * msk)

        @pl.when(i == 0)
        def _():
            lp_ref[...] = jnp.zeros((1, 1), F32)
            ls_ref[...] = jnp.zeros((1, 1), F32)

        lp_ref[...] = lp_ref[...] + jnp.reshape(lpos, (1, 1))
        ls_ref[...] = ls_ref[...] + jnp.reshape(lseq, (1, 1))

    grid = LP // BL
    wfull = lambda shape: pl.BlockSpec(shape, lambda i: (0, 0))
    return pl.pallas_call(
        body,
        grid=(grid,),
        in_specs=[
            pl.BlockSpec((BL, HID), lambda i: (i, 0)),
            pl.BlockSpec((BL, 9), lambda i: (i, 0)),
            pl.BlockSpec((BL, 9), lambda i: (i, 0)),
            pl.BlockSpec((BL, 9), lambda i: (i, 0)),
            pl.BlockSpec((BL, K), lambda i: (i, 0)),
            pl.BlockSpec((BL, K), lambda i: (i, 0)),
            wfull((HID, K)),
            wfull((1, K)),
            wfull((1, 2)),
        ],
        out_specs=[
            pl.BlockSpec((1, 1), lambda i: (0, 0)),
            pl.BlockSpec((1, 1), lambda i: (0, 0)),
        ],
        out_shape=[
            jax.ShapeDtypeStruct((1, 1), F32),
            jax.ShapeDtypeStruct((1, 1), F32),
        ],
        interpret=interpret,
    )(h, x, pn, ep, ct, c0h, w_out, b_out, sc)


def _pad_rows(a, n):
    return jnp.concatenate(
        [a, jnp.zeros((n - a.shape[0],) + a.shape[1:], a.dtype)], axis=0)


def kernel(p_0, c_0, v_0, e, t, W_in, b_in, We1, be1, We2, be2, Wa, ba,
           Wx1, bx1, Wx2, Wh1, bh1, Wh2, bh2, W_out, b_out, position_scale):
    betas = jnp.linspace(1e-4, 0.02, NUM_STEPS)
    alphas = 1.0 - betas
    alpha_bars = jnp.cumprod(alphas)

    row = e[0, 0].astype(jnp.int32)
    col = e[1, 0].astype(jnp.int32)

    p_norm = p_0 / position_scale
    key = jax.random.key(42)
    k1, k2, k3 = jax.random.split(key, 3)
    ab = alpha_bars[t]
    c0f = jnp.sqrt(ab)[:, None, None]
    c1f = jnp.sqrt(1.0 - ab)[:, None, None]
    eps_p = jax.random.normal(k1, p_norm.shape, dtype=p_norm.dtype)
    p_noisy = c0f * p_norm + c1f * eps_p
    s_0 = jax.random.categorical(k2, jnp.log(c_0 + 1e-8), axis=-1)
    c0h = jax.nn.one_hot(s_0, K, dtype=F32)
    c_noisy = ab[:, None, None] * c0h + (1.0 - ab[:, None, None]) / K
    s_noisy = jax.random.categorical(k3, jnp.log(c_noisy + 1e-8), axis=-1)
    beta = betas[t]

    te = jnp.stack([beta, jnp.sin(beta), jnp.cos(beta)], axis=-1)
    b_eff = te @ W_in[20:23] + b_in[None, :]
    win_a = W_in[0:20]

    alpha_t = alphas[t]
    ab_prev = jnp.where(t > 0, alpha_bars[jnp.maximum(t - 1, 0)], 1.0)
    sc = jnp.stack([alpha_t[0], ab_prev[0]]).reshape(1, 2).astype(F32)

    cn = _pad_rows(c_noisy[0], LP)
    x0 = _pad_rows(p_noisy[0], LP)
    pn = x0
    ep = _pad_rows(eps_p[0], LP)
    ct = _pad_rows(jax.nn.one_hot(s_noisy[0], K, dtype=F32), LP)
    c0hp = _pad_rows(c0h[0], LP)

    zpad = jnp.zeros((EP - EN,), jnp.int32)
    gidx = jnp.concatenate([row, zpad, col, zpad]).reshape(32, NCHG, 128)
    sidx = jnp.concatenate(
        [row, jnp.full((EP - EN,), LN, jnp.int32)]).reshape(32, NCHS, 128)

    a_l = [We1[l, 0:20, :] for l in range(NL)]
    b_l = [We1[l, 20:40, :] for l in range(NL)]
    we1c_l = [We1[l, 40:41, :] for l in range(NL)]
    be1_l = [be1[l][None, :] for l in range(NL)]
    be2_l = [be2[l][None, :] for l in range(NL)]
    wa_l = [Wa[l].T for l in range(NL)]
    ba_l = [ba[l].reshape(1, 1) for l in range(NL)]
    bx1_l = [bx1[l][None, :] for l in range(NL)]
    wx2_l = [Wx2[l].T for l in range(NL)]
    wh1a_l = [Wh1[l, 0:20, :] for l in range(NL)]
    wh1b_l = [Wh1[l, 20:40, :] for l in range(NL)]
    bh1_l = [bh1[l][None, :] for l in range(NL)]
    bh2_l = [bh2[l][None, :] for l in range(NL)]

    h, tr, tcn = _tc_init(cn, x0, win_a, b_eff, a_l[0], b_l[0])
    x = x0
    for l in range(NL):
        g = _sc_gather(tr, tcn, gidx)
        s4 = _tc_mlp(g.reshape(2 * EP // 4, 128), we1c_l[l], be1_l[l],
                     We2[l], be2_l[l], wa_l[l], ba_l[l], Wx1[l], bx1_l[l],
                     wx2_l[l])
        acc = _sc_scatter(s4.reshape(EP, 32), sidx)
        nxt = (l + 1) % NL
        h, x, tr, tcn = _tc_node(h, x, acc, wh1a_l[l], wh1b_l[l], bh1_l[l],
                                 Wh2[l], bh2_l[l], a_l[nxt], b_l[nxt])

    lp, ls = _tc_loss(h, x, pn, ep, ct, c0hp, W_out, b_out[None, :], sc)
    denom = jnp.float32(LN) + 1e-8
    return (lp[0, 0] / denom, ls[0, 0] / denom)


# speedup vs baseline: 13.6294x; 1.9758x over previous
import functools

import jax
import jax.numpy as jnp
from jax import lax
from jax.experimental import pallas as pl
from jax.experimental.pallas import tpu as pltpu
from jax.experimental.pallas import tpu_sc as plsc

NUM_STEPS = 100
K = 20
HID = 20
NL = 4

LN = 50000
LP = 50176
EN = 800000
EP = 802816
BL = 3584
BE = 8192
NCHG = 392
NCHS = 196
NB = 4
TPS = 3136
ZCH = 392
ZB_R = 196

F32 = jnp.float32


def _silu(x):
    return x * jax.nn.sigmoid(x)


def _sc_gather(tr, tc, gidx):
    mesh = plsc.VectorSubcoreMesh(core_axis_name="c", subcore_axis_name="s")

    def body(tr_ref, tc_ref, gi_ref, out_ref, idx_v, bufs, gsem, wsem):
        c_id = lax.axis_index("c")
        s_id = lax.axis_index("s")
        wid = c_id * 16 + s_id
        pltpu.sync_copy(gi_ref.at[wid], idx_v)
        base = wid * (NCHG * 128)

        def run(tbl):
            for b in range(NB):
                pltpu.async_copy(tbl.at[idx_v.at[b]], bufs.at[b], gsem.at[b])

            def step(g, carry):
                for b in range(NB):
                    c = g * NB + b
                    pltpu.make_async_copy(
                        tbl.at[idx_v.at[c]], bufs.at[b], gsem.at[b]).wait()
                    pltpu.async_copy(
                        bufs.at[b], out_ref.at[pl.ds(base + c * 128, 128)],
                        wsem.at[b])
                    nxt = c + NB

                    @pl.when(nxt < NCHG)
                    def _():
                        pltpu.make_async_copy(
                            bufs.at[b], out_ref.at[pl.ds(base, 128)],
                            wsem.at[b]).wait()
                        pltpu.async_copy(
                            tbl.at[idx_v.at[nxt]], bufs.at[b], gsem.at[b])
                return carry

            lax.fori_loop(0, NCHG // NB, step, 0)
            for b in range(NB):
                pltpu.make_async_copy(
                    bufs.at[b], out_ref.at[pl.ds(base, 128)], wsem.at[b]).wait()

        @pl.when(c_id == 0)
        def _():
            run(tr_ref)

        @pl.when(c_id == 1)
        def _():
            run(tc_ref)

    f = pl.kernel(
        body,
        out_type=jax.ShapeDtypeStruct((2 * EP, 32), F32),
        mesh=mesh,
        compiler_params=pltpu.CompilerParams(use_tc_tiling_on_sc=False),
        scratch_types=[
            pltpu.VMEM((NCHG, 128), jnp.int32),
            pltpu.VMEM((NB, 128, 32), F32),
            pltpu.SemaphoreType.DMA((NB,)),
            pltpu.SemaphoreType.DMA((NB,)),
        ],
    )
    return f(tr, tc, gidx)


def _sc_scatter(s_vals, sidx):
    mesh = plsc.VectorSubcoreMesh(core_axis_name="c", subcore_axis_name="s")

    def body(s_ref, si_ref, out_ref, acc, iring, vals, zb, lsem, isem):
        c_id = lax.axis_index("c")
        s_id = lax.axis_index("s")
        wid = c_id * 16 + s_id

        def zfill(r, carry):
            zb[r, pl.ds(0, 16)] = jnp.zeros((16,), F32)
            zb[r, pl.ds(16, 16)] = jnp.zeros((16,), F32)
            return carry

        lax.fori_loop(0, ZB_R, zfill, 0)
        for i in range(TPS // ZB_R):
            pltpu.sync_copy(zb, acc.at[pl.ds(s_id * TPS + i * ZB_R, ZB_R)])
        plsc.subcore_barrier()

        base = wid * (NCHS * 128)
        for b in range(NB):
            pltpu.async_copy(
                s_ref.at[pl.ds(base + b * 128, 128)], vals.at[b], lsem.at[b])
            pltpu.async_copy(si_ref.at[wid, b], iring.at[b], isem.at[b])

        def step(g, carry):
            for b in range(NB):
                c = g * NB + b
                pltpu.make_async_copy(
                    s_ref.at[pl.ds(base, 128)], vals.at[b], lsem.at[b]).wait()
                pltpu.make_async_copy(
                    si_ref.at[wid, c], iring.at[b], isem.at[b]).wait()
                pltpu.sync_copy(vals.at[b], acc.at[iring.at[b]], add=True)
                nxt = c + NB

                @pl.when(nxt < NCHS)
                def _():
                    pltpu.async_copy(
                        s_ref.at[pl.ds(base + nxt * 128, 128)], vals.at[b],
                        lsem.at[b])
                    pltpu.async_copy(si_ref.at[wid, nxt], iring.at[b],
                                     isem.at[b])
            return carry

        lax.fori_loop(0, NCHS // NB, step, 0)
        plsc.subcore_barrier()
        for i in range(TPS // ZCH):
            off = s_id * TPS + i * ZCH
            pltpu.sync_copy(acc.at[pl.ds(off, ZCH)],
                            out_ref.at[c_id, pl.ds(off, ZCH)])

    f = pl.kernel(
        body,
        out_type=jax.ShapeDtypeStruct((2, LP, 32), F32),
        mesh=mesh,
        compiler_params=pltpu.CompilerParams(use_tc_tiling_on_sc=False),
        scratch_types=[
            pltpu.VMEM_SHARED((LP, 32), F32),
            pltpu.VMEM((NB, 128), jnp.int32),
            pltpu.VMEM((NB, 128, 32), F32),
            pltpu.VMEM((ZB_R, 32), F32),
            pltpu.SemaphoreType.DMA((NB,)),
            pltpu.SemaphoreType.DMA((NB,)),
        ],
    )
    return f(s_vals, sidx)


def _tc_init(cn, x0, win_a, b_eff, a0, b0, interpret=False):
    def body(cn_ref, x_ref, wa_ref, be_ref, a_ref, b_ref,
             h_ref, tr_ref, tc_ref):
        h = jnp.dot(cn_ref[...], wa_ref[...],
                    preferred_element_type=F32) + be_ref[...]
        x = x_ref[...]
        p = jnp.dot(h, a_ref[...], preferred_element_type=F32)
        q = jnp.dot(h, b_ref[...], preferred_element_type=F32)
        z = jnp.zeros((BL, 3), F32)
        h_ref[...] = h
        tr_ref[...] = jnp.concatenate([p, x, z], axis=-1)
        tc_ref[...] = jnp.concatenate([q, x, z], axis=-1)

    grid = LP // BL
    return pl.pallas_call(
        body,
        grid=(grid,),
        in_specs=[
            pl.BlockSpec((BL, HID), lambda i: (i, 0)),
            pl.BlockSpec((BL, 9), lambda i: (i, 0)),
            pl.BlockSpec((HID, HID), lambda i: (0, 0)),
            pl.BlockSpec((1, HID), lambda i: (0, 0)),
            pl.BlockSpec((HID, HID), lambda i: (0, 0)),
            pl.BlockSpec((HID, HID), lambda i: (0, 0)),
        ],
        out_specs=[
            pl.BlockSpec((BL, HID), lambda i: (i, 0)),
            pl.BlockSpec((BL, 32), lambda i: (i, 0)),
            pl.BlockSpec((BL, 32), lambda i: (i, 0)),
        ],
        out_shape=[
            jax.ShapeDtypeStruct((LP, HID), F32),
            jax.ShapeDtypeStruct((LP, 32), F32),
            jax.ShapeDtypeStruct((LP, 32), F32),
        ],
        interpret=interpret,
    )(cn, x0, win_a, b_eff, a0, b0)


def _mlp_packed_weights(we1c, be1, we2, be2, wa, ba_s, wx1, bx1, wx2):
    off = jnp.arange(128) % 32
    grp = jnp.arange(128) // 32
    sameg = (grp[:, None] == grp[None, :]).astype(F32)
    mm = (off < 20).astype(F32)
    mt = ((off >= 20) & (off < 29)).astype(F32)
    mc = (off == 29).astype(F32)
    offc = jnp.clip(off, 0, 19)
    we1c_e = jnp.where(off < 20, we1c[0][offc], 0.0)
    wa_e = jnp.where(off < 20, wa[:, 0][offc], 0.0)
    wx2_e = jnp.where(off < 20, wx2[:, 0][offc], 0.0)
    w_rad = sameg * mt[:, None] * mm[None, :] * we1c_e[None, :]
    w_att = sameg * mm[:, None] * mm[None, :] * wa_e[:, None]
    w_wb = sameg * mm[:, None] * mt[None, :] * wx2_e[:, None]
    blk = sameg * mm[:, None] * mm[None, :]
    w2_pk = blk * we2[offc][:, offc]
    wx1_pk = blk * wx1[offc][:, offc]
    b1_t = (jnp.where(off < 20, be1[offc], 0.0))[None, :]
    b2_t = (jnp.where(off < 20, be2[offc], 0.0))[None, :]
    ba_t = (jnp.where(off < 20, ba_s, 0.0))[None, :]
    bx1_t = (jnp.where(off < 20, bx1[offc], 0.0))[None, :]
    return (w_rad, w2_pk, w_att, wx1_pk, w_wb, b1_t, b2_t, ba_t, bx1_t,
            mm[None, :], mc[None, :])


def _tc_mlp(g4, pk, interpret=False):
    B4 = BE // 4
    (w_rad, w2_pk, w_att, wx1_pk, w_wb, b1_t, b2_t, ba_t, bx1_t,
     mm, mc) = pk

    def body(gr_ref, gc_ref, wr_ref, w2_ref, watt_ref, wx1_ref, wwb_ref,
             b1_ref, b2_ref, ba_ref, bx1_ref, mm_ref, mc_ref, s_ref):
        gr = gr_ref[...]
        gc = gc_ref[...]
        d = gr - gc
        m1 = (gr + gc) * mm_ref[...] + jnp.dot(
            d * d, wr_ref[...], preferred_element_type=F32) + b1_ref[...]
        m1 = _silu(m1)
        m2 = _silu(jnp.dot(m1, w2_ref[...],
                           preferred_element_type=F32) + b2_ref[...])
        att = jax.nn.sigmoid(jnp.dot(m2, watt_ref[...],
                                     preferred_element_type=F32) + ba_ref[...])
        m = m2 * att
        u = _silu(jnp.dot(m, wx1_ref[...],
                          preferred_element_type=F32) + bx1_ref[...])
        wb = jnp.dot(u, wwb_ref[...], preferred_element_type=F32)
        s_ref[...] = m * mm_ref[...] + d * wb + mc_ref[...]

    grid = EP // BE
    wfull = lambda shape: pl.BlockSpec(shape, lambda i: (0, 0))
    return pl.pallas_call(
        body,
        grid=(grid,),
        in_specs=[
            pl.BlockSpec((B4, 128), lambda i: (i, 0)),
            pl.BlockSpec((B4, 128), lambda i: (i + EP // BE, 0)),
            wfull((128, 128)),
            wfull((128, 128)),
            wfull((128, 128)),
            wfull((128, 128)),
            wfull((128, 128)),
            wfull((1, 128)),
            wfull((1, 128)),
            wfull((1, 128)),
            wfull((1, 128)),
            wfull((1, 128)),
            wfull((1, 128)),
        ],
        out_specs=pl.BlockSpec((B4, 128), lambda i: (i, 0)),
        out_shape=jax.ShapeDtypeStruct((EP // 4, 128), F32),
        interpret=interpret,
    )(g4, g4, w_rad, w2_pk, w_att, wx1_pk, w_wb, b1_t, b2_t, ba_t, bx1_t,
      mm, mc)


def _tc_node(h, x, acc, wh1a, wh1b, bh1, wh2, bh2, a_nxt, b_nxt,
             interpret=False):
    def body(h_ref, x_ref, a0_ref, a1_ref, wh1a_ref, wh1b_ref, bh1_ref,
             wh2_ref, bh2_ref, an_ref, bn_ref,
             ho_ref, xo_ref, tro_ref, tco_ref):
        accs = a0_ref[0] + a1_ref[0]
        agg_m = accs[:, 0:20]
        cnt = jnp.maximum(accs[:, 29:30], 1.0)
        agg_x = accs[:, 20:29] / cnt
        x_new = x_ref[...] + agg_x
        h_old = h_ref[...]
        pre = jnp.dot(h_old, wh1a_ref[...], preferred_element_type=F32)
        pre = pre + jnp.dot(agg_m, wh1b_ref[...], preferred_element_type=F32)
        hn = jnp.dot(_silu(pre + bh1_ref[...]), wh2_ref[...],
                     preferred_element_type=F32) + bh2_ref[...]
        h_new = h_old + hn
        p = jnp.dot(h_new, an_ref[...], preferred_element_type=F32)
        q = jnp.dot(h_new, bn_ref[...], preferred_element_type=F32)
        z = jnp.zeros((BL, 3), F32)
        ho_ref[...] = h_new
        xo_ref[...] = x_new
        tro_ref[...] = jnp.concatenate([p, x_new, z], axis=-1)
        tco_ref[...] = jnp.concatenate([q, x_new, z], axis=-1)

    grid = LP // BL
    wfull = lambda shape: pl.BlockSpec(shape, lambda i: (0, 0))
    return pl.pallas_call(
        body,
        grid=(grid,),
        in_specs=[
            pl.BlockSpec((BL, HID), lambda i: (i, 0)),
            pl.BlockSpec((BL, 9), lambda i: (i, 0)),
            pl.BlockSpec((1, BL, 32), lambda i: (0, i, 0)),
            pl.BlockSpec((1, BL, 32), lambda i: (1, i, 0)),
            wfull((HID, HID)),
            wfull((HID, HID)),
            wfull((1, HID)),
            wfull((HID, HID)),
            wfull((1, HID)),
            wfull((HID, HID)),
            wfull((HID, HID)),
        ],
        out_specs=[
            pl.BlockSpec((BL, HID), lambda i: (i, 0)),
            pl.BlockSpec((BL, 9), lambda i: (i, 0)),
            pl.BlockSpec((BL, 32), lambda i: (i, 0)),
            pl.BlockSpec((BL, 32), lambda i: (i, 0)),
        ],
        out_shape=[
            jax.ShapeDtypeStruct((LP, HID), F32),
            jax.ShapeDtypeStruct((LP, 9), F32),
            jax.ShapeDtypeStruct((LP, 32), F32),
            jax.ShapeDtypeStruct((LP, 32), F32),
        ],
        interpret=interpret,
    )(h, x, acc, acc, wh1a, wh1b, bh1, wh2, bh2, a_nxt, b_nxt)


def _tc_loss(h, x, pn, ep, ct, c0h, w_out, b_out, sc, interpret=False):
    def body(h_ref, x_ref, pn_ref, ep_ref, ct_ref, c0h_ref, wo_ref, bo_ref,
             sc_ref, lp_ref, ls_ref):
        i = pl.program_id(0)
        cd = jax.nn.softmax(
            jnp.dot(h_ref[...], wo_ref[...],
                    preferred_element_type=F32) + bo_ref[...], axis=-1)
        alpha = sc_ref[0, 0]
        abp = sc_ref[0, 1]
        fac = alpha * ct_ref[...] + (1.0 - alpha) / K
        tt = fac * (abp * c0h_ref[...] + (1.0 - abp) / K)
        tp = fac * (abp * cd + (1.0 - abp) / K)
        pt = tt / jnp.sum(tt, axis=-1, keepdims=True)
        pp = tp / jnp.sum(tp, axis=-1, keepdims=True)
        kl = jnp.sum(pt * jnp.log(pt) - pt * jnp.log(pp + 1e-8), axis=-1,
                     keepdims=True)
        rows = i * BL + lax.broadcasted_iota(jnp.int32, (BL, 1), 0)
        msk = (rows < LN).astype(F32)
        d = (x_ref[...] - pn_ref[...]) - ep_ref[...]
        lpos = jnp.sum(jnp.sum(d * d, axis=-1, keepdims=True) * msk)
        lseq = jnp.sum(kl * msk)

        @pl.when(i == 0)
        def _():
            lp_ref[...] = jnp.zeros((1, 1), F32)
            ls_ref[...] = jnp.zeros((1, 1), F32)

        lp_ref[...] = lp_ref[...] + jnp.reshape(lpos, (1, 1))
        ls_ref[...] = ls_ref[...] + jnp.reshape(lseq, (1, 1))

    grid = LP // BL
    wfull = lambda shape: pl.BlockSpec(shape, lambda i: (0, 0))
    return pl.pallas_call(
        body,
        grid=(grid,),
        in_specs=[
            pl.BlockSpec((BL, HID), lambda i: (i, 0)),
            pl.BlockSpec((BL, 9), lambda i: (i, 0)),
            pl.BlockSpec((BL, 9), lambda i: (i, 0)),
            pl.BlockSpec((BL, 9), lambda i: (i, 0)),
            pl.BlockSpec((BL, K), lambda i: (i, 0)),
            pl.BlockSpec((BL, K), lambda i: (i, 0)),
            wfull((HID, K)),
            wfull((1, K)),
            wfull((1, 2)),
        ],
        out_specs=[
            pl.BlockSpec((1, 1), lambda i: (0, 0)),
            pl.BlockSpec((1, 1), lambda i: (0, 0)),
        ],
        out_shape=[
            jax.ShapeDtypeStruct((1, 1), F32),
            jax.ShapeDtypeStruct((1, 1), F32),
        ],
        interpret=interpret,
    )(h, x, pn, ep, ct, c0h, w_out, b_out, sc)


def _pad_rows(a, n):
    return jnp.concatenate(
        [a, jnp.zeros((n - a.shape[0],) + a.shape[1:], a.dtype)], axis=0)


def kernel(p_0, c_0, v_0, e, t, W_in, b_in, We1, be1, We2, be2, Wa, ba,
           Wx1, bx1, Wx2, Wh1, bh1, Wh2, bh2, W_out, b_out, position_scale):
    betas = jnp.linspace(1e-4, 0.02, NUM_STEPS)
    alphas = 1.0 - betas
    alpha_bars = jnp.cumprod(alphas)

    row = e[0, 0].astype(jnp.int32)
    col = e[1, 0].astype(jnp.int32)

    p_norm = p_0 / position_scale
    key = jax.random.key(42)
    k1, k2, k3 = jax.random.split(key, 3)
    ab = alpha_bars[t]
    c0f = jnp.sqrt(ab)[:, None, None]
    c1f = jnp.sqrt(1.0 - ab)[:, None, None]
    eps_p = jax.random.normal(k1, p_norm.shape, dtype=p_norm.dtype)
    p_noisy = c0f * p_norm + c1f * eps_p
    s_0 = jax.random.categorical(k2, jnp.log(c_0 + 1e-8), axis=-1)
    c0h = jax.nn.one_hot(s_0, K, dtype=F32)
    c_noisy = ab[:, None, None] * c0h + (1.0 - ab[:, None, None]) / K
    s_noisy = jax.random.categorical(k3, jnp.log(c_noisy + 1e-8), axis=-1)
    beta = betas[t]

    te = jnp.stack([beta, jnp.sin(beta), jnp.cos(beta)], axis=-1)
    b_eff = te @ W_in[20:23] + b_in[None, :]
    win_a = W_in[0:20]

    alpha_t = alphas[t]
    ab_prev = jnp.where(t > 0, alpha_bars[jnp.maximum(t - 1, 0)], 1.0)
    sc = jnp.stack([alpha_t[0], ab_prev[0]]).reshape(1, 2).astype(F32)

    cn = _pad_rows(c_noisy[0], LP)
    x0 = _pad_rows(p_noisy[0], LP)
    pn = x0
    ep = _pad_rows(eps_p[0], LP)
    ct = _pad_rows(jax.nn.one_hot(s_noisy[0], K, dtype=F32), LP)
    c0hp = _pad_rows(c0h[0], LP)

    zpad = jnp.zeros((EP - EN,), jnp.int32)
    gidx = jnp.concatenate([row, zpad, col, zpad]).reshape(32, NCHG, 128)
    sidx = jnp.concatenate(
        [row, jnp.full((EP - EN,), LN, jnp.int32)]).reshape(32, NCHS, 128)

    a_l = [We1[l, 0:20, :] for l in range(NL)]
    b_l = [We1[l, 20:40, :] for l in range(NL)]
    we1c_l = [We1[l, 40:41, :] for l in range(NL)]
    be1_l = [be1[l][None, :] for l in range(NL)]
    be2_l = [be2[l][None, :] for l in range(NL)]
    wa_l = [Wa[l].T for l in range(NL)]
    ba_l = [ba[l].reshape(1, 1) for l in range(NL)]
    bx1_l = [bx1[l][None, :] for l in range(NL)]
    wx2_l = [Wx2[l].T for l in range(NL)]
    wh1a_l = [Wh1[l, 0:20, :] for l in range(NL)]
    wh1b_l = [Wh1[l, 20:40, :] for l in range(NL)]
    bh1_l = [bh1[l][None, :] for l in range(NL)]
    bh2_l = [bh2[l][None, :] for l in range(NL)]

    h, tr, tcn = _tc_init(cn, x0, win_a, b_eff, a_l[0], b_l[0])
    x = x0
    pk_l = [_mlp_packed_weights(we1c_l[l], be1[l], We2[l], be2[l], Wa[l],
                                ba[l][0], Wx1[l], bx1[l], Wx2[l])
            for l in range(NL)]

    for l in range(NL):
        g = _sc_gather(tr, tcn, gidx)
        s4 = _tc_mlp(g.reshape(2 * EP // 4, 128), pk_l[l])
        acc = _sc_scatter(s4.reshape(EP, 32), sidx)
        nxt = (l + 1) % NL
        h, x, tr, tcn = _tc_node(h, x, acc, wh1a_l[l], wh1b_l[l], bh1_l[l],
                                 Wh2[l], bh2_l[l], a_l[nxt], b_l[nxt])

    lp, ls = _tc_loss(h, x, pn, ep, ct, c0hp, W_out, b_out[None, :], sc)
    denom = jnp.float32(LN) + 1e-8
    return (lp[0, 0] / denom, ls[0, 0] / denom)
